# baseline (speedup 1.0000x reference)
import functools

import jax
import jax.numpy as jnp
from jax import lax
from jax.experimental import pallas as pl
from jax.experimental.pallas import tpu as pltpu
from jax.experimental.pallas import tpu_sc as plsc

_DIM = 4096
_NE = 64
_K = 8
_TOKENS = 32768
_FLAT = _TOKENS * _K

_NC = 2
_NS = 16
_NW = _NC * _NS
_L = 16
_CHUNK = _FLAT // _NW
_PER_LANE = _CHUNK // _L

_TOK_BLOCK = 1024


def _topk_body(x_ref, w_ref, scores_ref, idx_ref):
    x = x_ref[...]
    w = w_ref[...]
    s = lax.dot_general(x, w, (((1,), (1,)), ((), ())),
                        preferred_element_type=jnp.float32)
    m = jnp.max(s, axis=1, keepdims=True)
    e = jnp.exp(s - m)
    p = e / jnp.sum(e, axis=1, keepdims=True)
    iota = lax.broadcasted_iota(jnp.int32, p.shape, 1).astype(jnp.float32)
    vals = p
    scs = []
    ids = []
    for _ in range(_K):
        mk = jnp.max(vals, axis=1, keepdims=True)
        ik = jnp.min(jnp.where(vals == mk, iota, float(_NE)), axis=1,
                     keepdims=True)
        scs.append(mk)
        ids.append(ik)
        vals = jnp.where(iota == ik, -1.0, vals)
    scores_ref[...] = jnp.concatenate(scs, axis=1)
    idx_ref[...] = jnp.concatenate(ids, axis=1).astype(jnp.int32)


def _tc_topk(x, W):
    nblk = _TOKENS // _TOK_BLOCK
    return pl.pallas_call(
        _topk_body,
        grid=(nblk,),
        in_specs=[
            pl.BlockSpec((_TOK_BLOCK, _DIM), lambda i: (i, 0)),
            pl.BlockSpec((_NE, _DIM), lambda i: (0, 0)),
        ],
        out_specs=[
            pl.BlockSpec((_TOK_BLOCK, _K), lambda i: (i, 0)),
            pl.BlockSpec((_TOK_BLOCK, _K), lambda i: (i, 0)),
        ],
        out_shape=[
            jax.ShapeDtypeStruct((_TOKENS, _K), jnp.float32),
            jax.ShapeDtypeStruct((_TOKENS, _K), jnp.int32),
        ],
    )(x, W)


def _hist_body(keys_hbm, h_out, s_out, keys_v, hist_v, svec_v):
    c = lax.axis_index("c")
    s = lax.axis_index("s")
    w = s * _NC + c
    pltpu.sync_copy(keys_hbm.at[pl.ds(w * _CHUNK, _CHUNK)], keys_v)

    zeros16 = jnp.zeros((_L,), jnp.int32)
    for j in range(_NE * _L // _L):
        hist_v[pl.ds(j * _L, _L)] = zeros16

    lane = lax.broadcasted_iota(jnp.int32, (_L,), 0)
    base_idx = lane * _PER_LANE
    base_hist = lane * _NE
    ones = jnp.ones((_L,), jnp.int32)

    def body(t, carry):
        idx = base_idx + t
        k = plsc.load_gather(keys_v, [idx])
        plsc.addupdate_scatter(hist_v, [base_hist + k], ones)
        return carry

    lax.fori_loop(0, _PER_LANE, body, 0)

    for j in range(_NE // _L):
        acc = zeros16
        for l in range(_L):
            acc = acc + hist_v[pl.ds(l * _NE + j * _L, _L)]
        svec_v[pl.ds(j * _L, _L)] = acc

    pltpu.sync_copy(hist_v, h_out.at[pl.ds(w * _NE * _L, _NE * _L)])
    pltpu.sync_copy(svec_v, s_out.at[pl.ds(w * _NE, _NE)])


def _sc_hist(keys):
    mesh = plsc.VectorSubcoreMesh(core_axis_name="c", subcore_axis_name="s")
    f = pl.kernel(
        _hist_body,
        out_type=(
            jax.ShapeDtypeStruct((_NW * _NE * _L,), jnp.int32),
            jax.ShapeDtypeStruct((_NW * _NE,), jnp.int32),
        ),
        mesh=mesh,
        scratch_types=[
            pltpu.VMEM((_CHUNK,), jnp.int32),
            pltpu.VMEM((_NE * _L,), jnp.int32),
            pltpu.VMEM((_NE,), jnp.int32),
        ],
        compiler_params=pltpu.CompilerParams(needs_layout_passes=False),
    )
    return f(keys)


def _place_body(keys_hbm, scores_hbm, h_hbm, s_hbm,
                scores_out, tok_out, hist_out,
                keys_v, scores_v, outpos_v, tok_v, sall_v, h_v,
                counters_v, hist64_v, sem1, sem2):
    c = lax.axis_index("c")
    s = lax.axis_index("s")
    w = s * _NC + c
    pltpu.sync_copy(keys_hbm.at[pl.ds(w * _CHUNK, _CHUNK)], keys_v)
    pltpu.sync_copy(scores_hbm.at[pl.ds(w * _CHUNK, _CHUNK)], scores_v)
    pltpu.sync_copy(s_hbm, sall_v)
    pltpu.sync_copy(h_hbm.at[pl.ds(w * _NE * _L, _NE * _L)], h_v)

    zeros16 = jnp.zeros((_L,), jnp.int32)

    T = []
    C = []
    for j in range(_NE // _L):
        def acc_body(w2, carry, j=j):
            tj, cj = carry
            row = sall_v[pl.ds(w2 * _NE + j * _L, _L)]
            tj = tj + row
            cj = cj + jnp.where(w2 < w, row, zeros16)
            return (tj, cj)
        tj, cj = lax.fori_loop(0, _NW, acc_body, (zeros16, zeros16))
        T.append(tj)
        C.append(cj)

    G = []
    carry = jnp.zeros((), jnp.int32)
    for j in range(_NE // _L):
        cum = plsc.cumsum(T[j])
        G.append(cum - T[j] + carry)
        carry = carry + jnp.sum(T[j])

    @pl.when(w == 0)
    def _():
        for j in range(_NE // _L):
            hist64_v[pl.ds(j * _L, _L)] = T[j]
        pltpu.sync_copy(hist64_v, hist_out)

    acc = [G[j] + C[j] for j in range(_NE // _L)]
    for l in range(_L):
        for j in range(_NE // _L):
            counters_v[pl.ds(l * _NE + j * _L, _L)] = acc[j]
            acc[j] = acc[j] + h_v[pl.ds(l * _NE + j * _L, _L)]

    lane = lax.broadcasted_iota(jnp.int32, (_L,), 0)
    base_idx = lane * _PER_LANE
    base_hist = lane * _NE
    ones = jnp.ones((_L,), jnp.int32)
    gbase = w * _CHUNK

    def body(t, carry):
        idx = base_idx + t
        k = plsc.load_gather(keys_v, [idx])
        cidx = base_hist + k
        pos = plsc.load_gather(counters_v, [cidx])
        plsc.store_scatter(outpos_v, [idx], pos)
        plsc.addupdate_scatter(counters_v, [cidx], ones)
        tok = lax.shift_right_logical(gbase + idx, 3)
        plsc.store_scatter(tok_v, [idx], tok)
        return carry

    lax.fori_loop(0, _PER_LANE, body, 0)

    cp1 = pltpu.async_copy(scores_v, scores_out.at[outpos_v], sem1)
    cp2 = pltpu.async_copy(tok_v, tok_out.at[outpos_v], sem2)
    cp1.wait()
    cp2.wait()


def _sc_place(keys, scores_flat, h, s):
    mesh = plsc.VectorSubcoreMesh(core_axis_name="c", subcore_axis_name="s")
    f = pl.kernel(
        _place_body,
        out_type=(
            jax.ShapeDtypeStruct((_FLAT,), jnp.float32),
            jax.ShapeDtypeStruct((_FLAT,), jnp.int32),
            jax.ShapeDtypeStruct((_NE,), jnp.int32),
        ),
        mesh=mesh,
        scratch_types=[
            pltpu.VMEM((_CHUNK,), jnp.int32),
            pltpu.VMEM((_CHUNK,), jnp.float32),
            pltpu.VMEM((_CHUNK,), jnp.int32),
            pltpu.VMEM((_CHUNK,), jnp.int32),
            pltpu.VMEM((_NW * _NE,), jnp.int32),
            pltpu.VMEM((_NE * _L,), jnp.int32),
            pltpu.VMEM((_NE * _L,), jnp.int32),
            pltpu.VMEM((_NE,), jnp.int32),
            pltpu.SemaphoreType.DMA,
            pltpu.SemaphoreType.DMA,
        ],
        compiler_params=pltpu.CompilerParams(needs_layout_passes=False),
    )
    return f(keys, scores_flat, h, s)


def kernel(x, W):
    top_scores, sel = _tc_topk(x, W)
    keys = sel.reshape(-1)
    scores_flat = top_scores.reshape(-1)
    h, s = _sc_hist(keys)
    out_scores, out_tok, hist = _sc_place(keys, scores_flat, h, s)
    return out_scores, out_tok, hist


# speedup vs baseline: 3.3653x; 3.3653x over previous
import functools

import jax
import jax.numpy as jnp
from jax import lax
from jax.experimental import pallas as pl
from jax.experimental.pallas import tpu as pltpu
from jax.experimental.pallas import tpu_sc as plsc

_DIM = 4096
_NE = 64
_K = 8
_TOKENS = 32768
_FLAT = _TOKENS * _K

_NC = 2
_NS = 16
_NW = _NC * _NS
_L = 16
_CHUNK = _FLAT // _NW
_PER_LANE = _CHUNK // _L

_TOK_BLOCK = 1024


def _topk_body(x_ref, w_ref, scores_ref, idx_ref):
    x = x_ref[...]
    w = w_ref[...]
    s = lax.dot_general(x, w, (((1,), (1,)), ((), ())),
                        preferred_element_type=jnp.float32)
    m = jnp.max(s, axis=1, keepdims=True)
    e = jnp.exp(s - m)
    p = e / jnp.sum(e, axis=1, keepdims=True)
    iota = lax.broadcasted_iota(jnp.int32, p.shape, 1).astype(jnp.float32)
    vals = p
    scs = []
    ids = []
    for _ in range(_K):
        mk = jnp.max(vals, axis=1, keepdims=True)
        ik = jnp.min(jnp.where(vals == mk, iota, float(_NE)), axis=1,
                     keepdims=True)
        scs.append(mk)
        ids.append(ik)
        vals = jnp.where(iota == ik, -1.0, vals)
    scores_ref[...] = jnp.concatenate(scs, axis=1)
    idx_ref[...] = jnp.concatenate(ids, axis=1).astype(jnp.int32)


def _tc_topk(x, W):
    nblk = _TOKENS // _TOK_BLOCK
    return pl.pallas_call(
        _topk_body,
        grid=(nblk,),
        in_specs=[
            pl.BlockSpec((_TOK_BLOCK, _DIM), lambda i: (i, 0)),
            pl.BlockSpec((_NE, _DIM), lambda i: (0, 0)),
        ],
        out_specs=[
            pl.BlockSpec((_TOK_BLOCK, _K), lambda i: (i, 0)),
            pl.BlockSpec((_TOK_BLOCK, _K), lambda i: (i, 0)),
        ],
        out_shape=[
            jax.ShapeDtypeStruct((_TOKENS, _K), jnp.float32),
            jax.ShapeDtypeStruct((_TOKENS, _K), jnp.int32),
        ],
    )(x, W)


def _hist_body(keys_hbm, h_out, s_out, keys_v, hist_v, svec_v):
    c = lax.axis_index("c")
    s = lax.axis_index("s")
    w = s * _NC + c
    pltpu.sync_copy(keys_hbm.at[pl.ds(w * _CHUNK, _CHUNK)], keys_v)

    zeros16 = jnp.zeros((_L,), jnp.int32)
    for j in range(_NE * _L // _L):
        hist_v[pl.ds(j * _L, _L)] = zeros16

    lane = lax.broadcasted_iota(jnp.int32, (_L,), 0)
    base_idx = lane * _PER_LANE
    base_hist = lane * _NE
    ones = jnp.ones((_L,), jnp.int32)

    def body(t, carry):
        idx = base_idx + t
        k = plsc.load_gather(keys_v, [idx])
        plsc.addupdate_scatter(hist_v, [base_hist + k], ones)
        return carry

    lax.fori_loop(0, _PER_LANE, body, 0)

    for j in range(_NE // _L):
        acc = zeros16
        for l in range(_L):
            acc = acc + hist_v[pl.ds(l * _NE + j * _L, _L)]
        svec_v[pl.ds(j * _L, _L)] = acc

    pltpu.sync_copy(hist_v, h_out.at[pl.ds(w * _NE * _L, _NE * _L)])
    pltpu.sync_copy(svec_v, s_out.at[pl.ds(w * _NE, _NE)])


def _sc_hist(keys):
    mesh = plsc.VectorSubcoreMesh(core_axis_name="c", subcore_axis_name="s")
    f = pl.kernel(
        _hist_body,
        out_type=(
            jax.ShapeDtypeStruct((_NW * _NE * _L,), jnp.int32),
            jax.ShapeDtypeStruct((_NW * _NE,), jnp.int32),
        ),
        mesh=mesh,
        scratch_types=[
            pltpu.VMEM((_CHUNK,), jnp.int32),
            pltpu.VMEM((_NE * _L,), jnp.int32),
            pltpu.VMEM((_NE,), jnp.int32),
        ],
        compiler_params=pltpu.CompilerParams(needs_layout_passes=False),
    )
    return f(keys)


def _place_body(keys_hbm, scores_hbm, h_hbm, s_hbm,
                part_out, hist_out,
                keys_v, scores_v, outpos_v, tok_v, zeros_v, sall_v, h_v,
                counters_v, hist64_v, shs_sp, sht_sp):
    c = lax.axis_index("c")
    s = lax.axis_index("s")
    w = s * _NC + c
    pltpu.sync_copy(keys_hbm.at[pl.ds(w * _CHUNK, _CHUNK)], keys_v)
    pltpu.sync_copy(scores_hbm.at[pl.ds(w * _CHUNK, _CHUNK)], scores_v)
    pltpu.sync_copy(s_hbm, sall_v)
    pltpu.sync_copy(h_hbm.at[pl.ds(w * _NE * _L, _NE * _L)], h_v)

    slc = _FLAT // _NS
    zeros16_ = jnp.zeros((_L,), jnp.int32)

    def zbody(t, carry):
        zeros_v[pl.ds(t * _L, _L)] = zeros16_
        return carry

    lax.fori_loop(0, slc // _L, zbody, 0)
    pltpu.sync_copy(zeros_v, shs_sp.at[pl.ds(s * slc, slc)])
    pltpu.sync_copy(zeros_v, sht_sp.at[pl.ds(s * slc, slc)])

    zeros16 = jnp.zeros((_L,), jnp.int32)

    T = []
    C = []
    for j in range(_NE // _L):
        def acc_body(w2, carry, j=j):
            tj, cj = carry
            row = sall_v[pl.ds(w2 * _NE + j * _L, _L)]
            tj = tj + row
            cj = cj + jnp.where(w2 < w, row, zeros16)
            return (tj, cj)
        tj, cj = lax.fori_loop(0, _NW, acc_body, (zeros16, zeros16))
        T.append(tj)
        C.append(cj)

    G = []
    carry = jnp.zeros((), jnp.int32)
    for j in range(_NE // _L):
        cum = plsc.cumsum(T[j])
        G.append(cum - T[j] + carry)
        carry = carry + jnp.sum(T[j])

    @pl.when(w == 0)
    def _():
        for j in range(_NE // _L):
            hist64_v[pl.ds(j * _L, _L)] = T[j]
        pltpu.sync_copy(hist64_v, hist_out)

    acc = [G[j] + C[j] for j in range(_NE // _L)]
    for l in range(_L):
        for j in range(_NE // _L):
            counters_v[pl.ds(l * _NE + j * _L, _L)] = acc[j]
            acc[j] = acc[j] + h_v[pl.ds(l * _NE + j * _L, _L)]

    lane = lax.broadcasted_iota(jnp.int32, (_L,), 0)
    base_idx = lane * _PER_LANE
    base_hist = lane * _NE
    ones = jnp.ones((_L,), jnp.int32)
    gbase = w * _CHUNK

    def body(t, carry):
        idx = base_idx + t
        k = plsc.load_gather(keys_v, [idx])
        cidx = base_hist + k
        pos = plsc.load_gather(counters_v, [cidx])
        plsc.store_scatter(outpos_v, [idx], pos)
        plsc.addupdate_scatter(counters_v, [cidx], ones)
        return carry

    lax.fori_loop(0, _PER_LANE, body, 0)

    def tbody(t, carry):
        tok = lax.shift_right_logical(gbase + t * _L + lane, 3)
        tok_v[pl.ds(t * _L, _L)] = tok
        return carry

    lax.fori_loop(0, _PER_LANE, tbody, 0)

    pltpu.sync_copy(scores_v, shs_sp.at[outpos_v])
    pltpu.sync_copy(tok_v, sht_sp.at[outpos_v])
    plsc.subcore_barrier()

    pltpu.sync_copy(shs_sp.at[pl.ds(s * slc, slc)],
                    part_out.at[pl.ds(c * 2 * _FLAT + s * slc, slc)])
    pltpu.sync_copy(sht_sp.at[pl.ds(s * slc, slc)],
                    part_out.at[pl.ds(c * 2 * _FLAT + _FLAT + s * slc, slc)])


def _sc_place(keys, scores_bits, h, s):
    mesh = plsc.VectorSubcoreMesh(core_axis_name="c", subcore_axis_name="s")
    f = pl.kernel(
        _place_body,
        out_type=(
            jax.ShapeDtypeStruct((2 * 2 * _FLAT,), jnp.int32),
            jax.ShapeDtypeStruct((_NE,), jnp.int32),
        ),
        mesh=mesh,
        scratch_types=[
            pltpu.VMEM((_CHUNK,), jnp.int32),
            pltpu.VMEM((_CHUNK,), jnp.int32),
            pltpu.VMEM((_CHUNK,), jnp.int32),
            pltpu.VMEM((_CHUNK,), jnp.int32),
            pltpu.VMEM((_FLAT // _NS,), jnp.int32),
            pltpu.VMEM((_NW * _NE,), jnp.int32),
            pltpu.VMEM((_NE * _L,), jnp.int32),
            pltpu.VMEM((_NE * _L,), jnp.int32),
            pltpu.VMEM((_NE,), jnp.int32),
            pltpu.VMEM_SHARED((_FLAT,), jnp.int32),
            pltpu.VMEM_SHARED((_FLAT,), jnp.int32),
        ],
        compiler_params=pltpu.CompilerParams(needs_layout_passes=False),
    )
    return f(keys, scores_bits, h, s)


_CMB_BLK = 65536


def _combine_body(ps_ref, pt_ref, s_ref, t_ref):
    sb = ps_ref[0, :] | ps_ref[1, :]
    t_ref[...] = pt_ref[0, :] | pt_ref[1, :]
    s_ref[...] = lax.bitcast_convert_type(sb, jnp.float32)


def _tc_combine(ps, pt):
    nblk = _FLAT // _CMB_BLK
    return pl.pallas_call(
        _combine_body,
        grid=(nblk,),
        in_specs=[
            pl.BlockSpec((2, _CMB_BLK), lambda i: (0, i)),
            pl.BlockSpec((2, _CMB_BLK), lambda i: (0, i)),
        ],
        out_specs=[
            pl.BlockSpec((_CMB_BLK,), lambda i: (i,)),
            pl.BlockSpec((_CMB_BLK,), lambda i: (i,)),
        ],
        out_shape=[
            jax.ShapeDtypeStruct((_FLAT,), jnp.float32),
            jax.ShapeDtypeStruct((_FLAT,), jnp.int32),
        ],
    )(ps, pt)


def kernel(x, W):
    top_scores, sel = _tc_topk(x, W)
    keys = sel.reshape(-1)
    scores_bits = lax.bitcast_convert_type(top_scores.reshape(-1), jnp.int32)
    h, s = _sc_hist(keys)
    part, hist = _sc_place(keys, scores_bits, h, s)
    part = part.reshape(2, 2, _FLAT)
    ps = part[:, 0, :]
    pt = part[:, 1, :]
    out_scores, out_tok = _tc_combine(ps, pt)
    return out_scores, out_tok, hist


# speedup vs baseline: 3.4116x; 1.0138x over previous
import functools

import jax
import jax.numpy as jnp
from jax import lax
from jax.experimental import pallas as pl
from jax.experimental.pallas import tpu as pltpu
from jax.experimental.pallas import tpu_sc as plsc

_DIM = 4096
_NE = 64
_K = 8
_TOKENS = 32768
_FLAT = _TOKENS * _K

_NC = 2
_NS = 16
_NW = _NC * _NS
_L = 16
_CHUNK = _FLAT // _NW
_PER_LANE = _CHUNK // _L
_VC = 2
_VCHUNK = _CHUNK // _VC
_PLV = _VCHUNK // _L
_NV = _NW * _VC

_TOK_BLOCK = 1024


def _topk_body(x_ref, w_ref, scores_ref, idx_ref):
    x = x_ref[...]
    w = w_ref[...]
    s = lax.dot_general(x, w, (((1,), (1,)), ((), ())),
                        preferred_element_type=jnp.float32)
    m = jnp.max(s, axis=1, keepdims=True)
    e = jnp.exp(s - m)
    p = e / jnp.sum(e, axis=1, keepdims=True)
    iota = lax.broadcasted_iota(jnp.int32, p.shape, 1).astype(jnp.float32)
    vals = p
    scs = []
    ids = []
    for _ in range(_K):
        mk = jnp.max(vals, axis=1, keepdims=True)
        ik = jnp.min(jnp.where(vals == mk, iota, float(_NE)), axis=1,
                     keepdims=True)
        scs.append(mk)
        ids.append(ik)
        vals = jnp.where(iota == ik, -1.0, vals)
    scores_ref[...] = jnp.concatenate(scs, axis=1)
    idx_ref[...] = jnp.concatenate(ids, axis=1).astype(jnp.int32)


def _tc_topk(x, W):
    nblk = _TOKENS // _TOK_BLOCK
    return pl.pallas_call(
        _topk_body,
        grid=(nblk,),
        in_specs=[
            pl.BlockSpec((_TOK_BLOCK, _DIM), lambda i: (i, 0)),
            pl.BlockSpec((_NE, _DIM), lambda i: (0, 0)),
        ],
        out_specs=[
            pl.BlockSpec((_TOK_BLOCK, _K), lambda i: (i, 0)),
            pl.BlockSpec((_TOK_BLOCK, _K), lambda i: (i, 0)),
        ],
        out_shape=[
            jax.ShapeDtypeStruct((_TOKENS, _K), jnp.float32),
            jax.ShapeDtypeStruct((_TOKENS, _K), jnp.int32),
        ],
    )(x, W)


def _hist_body(keys_hbm, h_out, s_out, keys_v, hist_v, svec_v):
    c = lax.axis_index("c")
    s = lax.axis_index("s")
    w = s * _NC + c
    pltpu.sync_copy(keys_hbm.at[pl.ds(w * _CHUNK, _CHUNK)], keys_v)

    zeros16 = jnp.zeros((_L,), jnp.int32)
    for j in range(_VC * _NE * _L // _L):
        hist_v[pl.ds(j * _L, _L)] = zeros16

    lane = lax.broadcasted_iota(jnp.int32, (_L,), 0)
    base_idx = lane * _PLV
    base_hist = lane * _NE
    ones = jnp.ones((_L,), jnp.int32)

    def body(t, carry):
        for u in range(_VC):
            idx = base_idx + (u * _VCHUNK + t)
            k = plsc.load_gather(keys_v, [idx])
            plsc.addupdate_scatter(hist_v, [(u * _NE * _L) + base_hist + k],
                                   ones)
        return carry

    lax.fori_loop(0, _PLV, body, 0)

    for u in range(_VC):
        for j in range(_NE // _L):
            acc = zeros16
            for l in range(_L):
                acc = acc + hist_v[pl.ds(u * _NE * _L + l * _NE + j * _L, _L)]
            svec_v[pl.ds(u * _NE + j * _L, _L)] = acc

    pltpu.sync_copy(hist_v, h_out.at[pl.ds(w * _VC * _NE * _L, _VC * _NE * _L)])
    pltpu.sync_copy(svec_v, s_out.at[pl.ds(w * _VC * _NE, _VC * _NE)])


def _sc_hist(keys):
    mesh = plsc.VectorSubcoreMesh(core_axis_name="c", subcore_axis_name="s")
    f = pl.kernel(
        _hist_body,
        out_type=(
            jax.ShapeDtypeStruct((_NV * _NE * _L,), jnp.int32),
            jax.ShapeDtypeStruct((_NV * _NE,), jnp.int32),
        ),
        mesh=mesh,
        scratch_types=[
            pltpu.VMEM((_CHUNK,), jnp.int32),
            pltpu.VMEM((_VC * _NE * _L,), jnp.int32),
            pltpu.VMEM((_VC * _NE,), jnp.int32),
        ],
        compiler_params=pltpu.CompilerParams(needs_layout_passes=False),
    )
    return f(keys)


def _place_body(keys_hbm, scores_hbm, h_hbm, s_hbm,
                ps_out, pt_out, hist_out,
                keys_v, scores_v, outpos_v, tok_v, zeros_v, sall_v, h_v,
                counters_v, hist64_v, shs_sp, sht_sp):
    c = lax.axis_index("c")
    s = lax.axis_index("s")
    w = s * _NC + c
    pltpu.sync_copy(keys_hbm.at[pl.ds(w * _CHUNK, _CHUNK)], keys_v)
    pltpu.sync_copy(scores_hbm.at[pl.ds(w * _CHUNK, _CHUNK)], scores_v)
    pltpu.sync_copy(s_hbm, sall_v)
    pltpu.sync_copy(h_hbm.at[pl.ds(w * _VC * _NE * _L, _VC * _NE * _L)], h_v)

    slc = _FLAT // _NS
    zeros16_ = jnp.zeros((_L,), jnp.int32)

    def zbody(t, carry):
        zeros_v[pl.ds(t * _L, _L)] = zeros16_
        return carry

    lax.fori_loop(0, slc // _L, zbody, 0)
    pltpu.sync_copy(zeros_v, shs_sp.at[pl.ds(s * slc, slc)])
    pltpu.sync_copy(zeros_v, sht_sp.at[pl.ds(s * slc, slc)])

    zeros16 = jnp.zeros((_L,), jnp.int32)

    T = []
    C0 = []
    C1 = []
    for j in range(_NE // _L):
        def acc_body(v2, carry, j=j):
            tj, cj = carry
            row = sall_v[pl.ds(v2 * _NE + j * _L, _L)]
            tj = tj + row
            cj = cj + jnp.where(v2 < 2 * w, row, zeros16)
            return (tj, cj)
        tj, cj = lax.fori_loop(0, _NV, acc_body, (zeros16, zeros16))
        T.append(tj)
        C0.append(cj)
        C1.append(cj + sall_v[pl.ds((2 * w) * _NE + j * _L, _L)])

    G = []
    carry = jnp.zeros((), jnp.int32)
    for j in range(_NE // _L):
        cum = plsc.cumsum(T[j])
        G.append(cum - T[j] + carry)
        carry = carry + jnp.sum(T[j])

    @pl.when(w == 0)
    def _():
        for j in range(_NE // _L):
            hist64_v[pl.ds(j * _L, _L)] = T[j]
        pltpu.sync_copy(hist64_v, hist_out)

    for u, cu in ((0, C0), (1, C1)):
        acc = [G[j] + cu[j] for j in range(_NE // _L)]
        for l in range(_L):
            for j in range(_NE // _L):
                off = u * _NE * _L + l * _NE + j * _L
                counters_v[pl.ds(off, _L)] = acc[j]
                acc[j] = acc[j] + h_v[pl.ds(off, _L)]

    lane = lax.broadcasted_iota(jnp.int32, (_L,), 0)
    base_idx = lane * _PLV
    base_hist = lane * _NE
    ones = jnp.ones((_L,), jnp.int32)
    gbase = w * _CHUNK

    def body(t, carry):
        for u in range(_VC):
            idx = base_idx + (u * _VCHUNK + t)
            k = plsc.load_gather(keys_v, [idx])
            cidx = (u * _NE * _L) + base_hist + k
            pos = plsc.load_gather(counters_v, [cidx])
            plsc.store_scatter(outpos_v, [idx], pos)
            plsc.addupdate_scatter(counters_v, [cidx], ones)
        return carry

    lax.fori_loop(0, _PLV, body, 0)

    def tbody(t, carry):
        tok = lax.shift_right_logical(gbase + t * _L + lane, 3)
        tok_v[pl.ds(t * _L, _L)] = tok
        return carry

    lax.fori_loop(0, _PER_LANE, tbody, 0)

    plsc.subcore_barrier()
    pltpu.sync_copy(scores_v, shs_sp.at[outpos_v])
    pltpu.sync_copy(tok_v, sht_sp.at[outpos_v])
    plsc.subcore_barrier()

    pltpu.sync_copy(shs_sp.at[pl.ds(s * slc, slc)],
                    ps_out.at[pl.ds(c * _FLAT + s * slc, slc)])
    pltpu.sync_copy(sht_sp.at[pl.ds(s * slc, slc)],
                    pt_out.at[pl.ds(c * _FLAT + s * slc, slc)])


def _sc_place(keys, scores_bits, h, s):
    mesh = plsc.VectorSubcoreMesh(core_axis_name="c", subcore_axis_name="s")
    f = pl.kernel(
        _place_body,
        out_type=(
            jax.ShapeDtypeStruct((2 * _FLAT,), jnp.int32),
            jax.ShapeDtypeStruct((2 * _FLAT,), jnp.int32),
            jax.ShapeDtypeStruct((_NE,), jnp.int32),
        ),
        mesh=mesh,
        scratch_types=[
            pltpu.VMEM((_CHUNK,), jnp.int32),
            pltpu.VMEM((_CHUNK,), jnp.int32),
            pltpu.VMEM((_CHUNK,), jnp.int32),
            pltpu.VMEM((_CHUNK,), jnp.int32),
            pltpu.VMEM((_FLAT // _NS,), jnp.int32),
            pltpu.VMEM((_NV * _NE,), jnp.int32),
            pltpu.VMEM((_VC * _NE * _L,), jnp.int32),
            pltpu.VMEM((_VC * _NE * _L,), jnp.int32),
            pltpu.VMEM((_NE,), jnp.int32),
            pltpu.VMEM_SHARED((_FLAT,), jnp.int32),
            pltpu.VMEM_SHARED((_FLAT,), jnp.int32),
        ],
        compiler_params=pltpu.CompilerParams(needs_layout_passes=False),
    )
    return f(keys, scores_bits, h, s)


_CMB_BLK = 65536


def _combine_body(ps_ref, pt_ref, s_ref, t_ref):
    sb = ps_ref[0, :] | ps_ref[1, :]
    t_ref[...] = pt_ref[0, :] | pt_ref[1, :]
    s_ref[...] = lax.bitcast_convert_type(sb, jnp.float32)


def _tc_combine(ps, pt):
    nblk = _FLAT // _CMB_BLK
    return pl.pallas_call(
        _combine_body,
        grid=(nblk,),
        in_specs=[
            pl.BlockSpec((2, _CMB_BLK), lambda i: (0, i)),
            pl.BlockSpec((2, _CMB_BLK), lambda i: (0, i)),
        ],
        out_specs=[
            pl.BlockSpec((_CMB_BLK,), lambda i: (i,)),
            pl.BlockSpec((_CMB_BLK,), lambda i: (i,)),
        ],
        out_shape=[
            jax.ShapeDtypeStruct((_FLAT,), jnp.float32),
            jax.ShapeDtypeStruct((_FLAT,), jnp.int32),
        ],
    )(ps, pt)


def kernel(x, W):
    top_scores, sel = _tc_topk(x, W)
    keys = sel.reshape(-1)
    scores_bits = lax.bitcast_convert_type(top_scores.reshape(-1), jnp.int32)
    h, s = _sc_hist(keys)
    ps, pt, hist = _sc_place(keys, scores_bits, h, s)
    out_scores, out_tok = _tc_combine(ps.reshape(2, _FLAT), pt.reshape(2, _FLAT))
    return out_scores, out_tok, hist


# speedup vs baseline: 3.5221x; 1.0324x over previous
import functools

import jax
import jax.numpy as jnp
from jax import lax
from jax.experimental import pallas as pl
from jax.experimental.pallas import tpu as pltpu
from jax.experimental.pallas import tpu_sc as plsc

_DIM = 4096
_NE = 64
_K = 8
_TOKENS = 32768
_FLAT = _TOKENS * _K

_NC = 2
_NS = 16
_NW = _NC * _NS
_L = 16
_CHUNK = _FLAT // _NW
_PER_LANE = _CHUNK // _L
_VC = 2
_VCHUNK = _CHUNK // _VC
_PLV = _VCHUNK // _L
_NV = _NW * _VC

_TOK_BLOCK = 1024


def _topk_body(x_ref, w_ref, scores_ref, idx_ref):
    x = x_ref[...]
    w = w_ref[...]
    s = lax.dot_general(x, w, (((1,), (1,)), ((), ())),
                        preferred_element_type=jnp.float32)
    m = jnp.max(s, axis=1, keepdims=True)
    e = jnp.exp(s - m)
    p = e / jnp.sum(e, axis=1, keepdims=True)
    iota = lax.broadcasted_iota(jnp.int32, p.shape, 1).astype(jnp.float32)
    vals = p
    scs = []
    ids = []
    for _ in range(_K):
        mk = jnp.max(vals, axis=1, keepdims=True)
        ik = jnp.min(jnp.where(vals == mk, iota, float(_NE)), axis=1,
                     keepdims=True)
        scs.append(mk)
        ids.append(ik)
        vals = jnp.where(iota == ik, -1.0, vals)
    scores_ref[...] = jnp.concatenate(scs, axis=1)
    idx_ref[...] = jnp.concatenate(ids, axis=1).astype(jnp.int32)


def _tc_topk(x, W):
    nblk = _TOKENS // _TOK_BLOCK
    return pl.pallas_call(
        _topk_body,
        grid=(nblk,),
        in_specs=[
            pl.BlockSpec((_TOK_BLOCK, _DIM), lambda i: (i, 0)),
            pl.BlockSpec((_NE, _DIM), lambda i: (0, 0)),
        ],
        out_specs=[
            pl.BlockSpec((_TOK_BLOCK, _K), lambda i: (i, 0)),
            pl.BlockSpec((_TOK_BLOCK, _K), lambda i: (i, 0)),
        ],
        out_shape=[
            jax.ShapeDtypeStruct((_TOKENS, _K), jnp.float32),
            jax.ShapeDtypeStruct((_TOKENS, _K), jnp.int32),
        ],
    )(x, W)


def _hist_body(keys_hbm, h_out, s_out, keys_v, hist_v, svec_v):
    c = lax.axis_index("c")
    s = lax.axis_index("s")
    w = s * _NC + c
    pltpu.sync_copy(keys_hbm.at[pl.ds(w * _CHUNK, _CHUNK)], keys_v)

    zeros16 = jnp.zeros((_L,), jnp.int32)
    for j in range(_VC * _NE * _L // _L):
        hist_v[pl.ds(j * _L, _L)] = zeros16

    lane = lax.broadcasted_iota(jnp.int32, (_L,), 0)
    base_idx = lane * _PLV
    base_hist = lane * _NE
    ones = jnp.ones((_L,), jnp.int32)

    def body(t, carry):
        for u in range(_VC):
            idx = base_idx + (u * _VCHUNK + t)
            k = plsc.load_gather(keys_v, [idx])
            plsc.addupdate_scatter(hist_v, [(u * _NE * _L) + base_hist + k],
                                   ones)
        return carry

    lax.fori_loop(0, _PLV, body, 0)

    for u in range(_VC):
        for j in range(_NE // _L):
            acc = zeros16
            for l in range(_L):
                acc = acc + hist_v[pl.ds(u * _NE * _L + l * _NE + j * _L, _L)]
            svec_v[pl.ds(u * _NE + j * _L, _L)] = acc

    pltpu.sync_copy(hist_v, h_out.at[pl.ds(w * _VC * _NE * _L, _VC * _NE * _L)])
    pltpu.sync_copy(svec_v, s_out.at[pl.ds(w * _VC * _NE, _VC * _NE)])


def _sc_hist(keys):
    mesh = plsc.VectorSubcoreMesh(core_axis_name="c", subcore_axis_name="s")
    f = pl.kernel(
        _hist_body,
        out_type=(
            jax.ShapeDtypeStruct((_NV * _NE * _L,), jnp.int32),
            jax.ShapeDtypeStruct((_NV * _NE,), jnp.int32),
        ),
        mesh=mesh,
        scratch_types=[
            pltpu.VMEM((_CHUNK,), jnp.int32),
            pltpu.VMEM((_VC * _NE * _L,), jnp.int32),
            pltpu.VMEM((_VC * _NE,), jnp.int32),
        ],
        compiler_params=pltpu.CompilerParams(needs_layout_passes=False),
    )
    return f(keys)


_VCP = 4
_CHUNK_P = _FLAT // _NS


def _place_body(keys_hbm, scores_hbm, h_hbm, s_hbm,
                scores_out, tok_out, hist_out,
                keys_v, scores_v, outpos_v, tok_v, sall_v, h_v,
                counters_v, hist64_v, shs_sp, sht_sp):
    c = lax.axis_index("c")
    s = lax.axis_index("s")

    @pl.when(c == 0)
    def _():
        pltpu.sync_copy(keys_hbm.at[pl.ds(s * _CHUNK_P, _CHUNK_P)], keys_v)
        pltpu.sync_copy(scores_hbm.at[pl.ds(s * _CHUNK_P, _CHUNK_P)], scores_v)
        pltpu.sync_copy(s_hbm, sall_v)
        pltpu.sync_copy(h_hbm.at[pl.ds(s * _VCP * _NE * _L, _VCP * _NE * _L)],
                        h_v)

        zeros16 = jnp.zeros((_L,), jnp.int32)

        T = []
        Cu = [[] for _ in range(_VCP)]
        for j in range(_NE // _L):
            def acc_body(v2, carry, j=j):
                tj, cj = carry
                row = sall_v[pl.ds(v2 * _NE + j * _L, _L)]
                tj = tj + row
                cj = cj + jnp.where(v2 < _VCP * s, row, zeros16)
                return (tj, cj)
            tj, cj = lax.fori_loop(0, _NV, acc_body, (zeros16, zeros16))
            T.append(tj)
            Cu[0].append(cj)
            for u in range(1, _VCP):
                cj = cj + sall_v[pl.ds((_VCP * s + u - 1) * _NE + j * _L, _L)]
                Cu[u].append(cj)

        G = []
        carry = jnp.zeros((), jnp.int32)
        for j in range(_NE // _L):
            cum = plsc.cumsum(T[j])
            G.append(cum - T[j] + carry)
            carry = carry + jnp.sum(T[j])

        @pl.when(s == 0)
        def _():
            for j in range(_NE // _L):
                hist64_v[pl.ds(j * _L, _L)] = T[j]
            pltpu.sync_copy(hist64_v, hist_out)

        for u in range(_VCP):
            acc = [G[j] + Cu[u][j] for j in range(_NE // _L)]
            for l in range(_L):
                for j in range(_NE // _L):
                    off = u * _NE * _L + l * _NE + j * _L
                    counters_v[pl.ds(off, _L)] = acc[j]
                    acc[j] = acc[j] + h_v[pl.ds(off, _L)]

        lane = lax.broadcasted_iota(jnp.int32, (_L,), 0)
        base_idx = lane * _PLV
        base_hist = lane * _NE
        ones = jnp.ones((_L,), jnp.int32)
        gbase = s * _CHUNK_P

        def body(t, carry):
            for u in range(_VCP):
                idx = base_idx + (u * _VCHUNK + t)
                k = plsc.load_gather(keys_v, [idx])
                cidx = (u * _NE * _L) + base_hist + k
                pos = plsc.load_gather(counters_v, [cidx])
                plsc.store_scatter(outpos_v, [idx], pos)
                plsc.addupdate_scatter(counters_v, [cidx], ones)
            return carry

        lax.fori_loop(0, _PLV, body, 0)

        def tbody(t, carry):
            tok = lax.shift_right_logical(gbase + t * _L + lane, 3)
            tok_v[pl.ds(t * _L, _L)] = tok
            return carry

        lax.fori_loop(0, _CHUNK_P // _L, tbody, 0)

        pltpu.sync_copy(scores_v, shs_sp.at[outpos_v])
        pltpu.sync_copy(tok_v, sht_sp.at[outpos_v])
        plsc.subcore_barrier()

        pltpu.sync_copy(shs_sp.at[pl.ds(s * _CHUNK_P, _CHUNK_P)],
                        scores_out.at[pl.ds(s * _CHUNK_P, _CHUNK_P)])
        pltpu.sync_copy(sht_sp.at[pl.ds(s * _CHUNK_P, _CHUNK_P)],
                        tok_out.at[pl.ds(s * _CHUNK_P, _CHUNK_P)])


def _sc_place(keys, scores_flat, h, s):
    mesh = plsc.VectorSubcoreMesh(core_axis_name="c", subcore_axis_name="s")
    f = pl.kernel(
        _place_body,
        out_type=(
            jax.ShapeDtypeStruct((_FLAT,), jnp.float32),
            jax.ShapeDtypeStruct((_FLAT,), jnp.int32),
            jax.ShapeDtypeStruct((_NE,), jnp.int32),
        ),
        mesh=mesh,
        scratch_types=[
            pltpu.VMEM((_CHUNK_P,), jnp.int32),
            pltpu.VMEM((_CHUNK_P,), jnp.float32),
            pltpu.VMEM((_CHUNK_P,), jnp.int32),
            pltpu.VMEM((_CHUNK_P,), jnp.int32),
            pltpu.VMEM((_NV * _NE,), jnp.int32),
            pltpu.VMEM((_VCP * _NE * _L,), jnp.int32),
            pltpu.VMEM((_VCP * _NE * _L,), jnp.int32),
            pltpu.VMEM((_NE,), jnp.int32),
            pltpu.VMEM_SHARED((_FLAT,), jnp.float32),
            pltpu.VMEM_SHARED((_FLAT,), jnp.int32),
        ],
        compiler_params=pltpu.CompilerParams(needs_layout_passes=False),
    )
    return f(keys, scores_flat, h, s)


def kernel(x, W):
    top_scores, sel = _tc_topk(x, W)
    keys = sel.reshape(-1)
    scores_flat = top_scores.reshape(-1)
    h, s = _sc_hist(keys)
    out_scores, out_tok, hist = _sc_place(keys, scores_flat, h, s)
    return out_scores, out_tok, hist


# speedup vs baseline: 3.6117x; 1.0254x over previous
import functools

import jax
import jax.numpy as jnp
from jax import lax
from jax.experimental import pallas as pl
from jax.experimental.pallas import tpu as pltpu
from jax.experimental.pallas import tpu_sc as plsc

_DIM = 4096
_NE = 64
_K = 8
_TOKENS = 32768
_FLAT = _TOKENS * _K

_NC = 2
_NS = 16
_NW = _NC * _NS
_L = 16
_CHUNK = _FLAT // _NW
_PER_LANE = _CHUNK // _L
_VC = 2
_VCHUNK = _CHUNK // _VC
_PLV = _VCHUNK // _L
_NV = _NW * _VC

_TOK_BLOCK = 1024


def _topk_body(x_ref, w_ref, scores_ref, idx_ref):
    x = x_ref[...]
    w = w_ref[...]
    s = lax.dot_general(x, w, (((1,), (1,)), ((), ())),
                        preferred_element_type=jnp.float32)
    m = jnp.max(s, axis=1, keepdims=True)
    e = jnp.exp(s - m)
    p = e / jnp.sum(e, axis=1, keepdims=True)
    iota = lax.broadcasted_iota(jnp.int32, p.shape, 1).astype(jnp.float32)
    vals = p
    scs = []
    ids = []
    for _ in range(_K):
        mk = jnp.max(vals, axis=1, keepdims=True)
        ik = jnp.min(jnp.where(vals == mk, iota, float(_NE)), axis=1,
                     keepdims=True)
        scs.append(mk)
        ids.append(ik)
        vals = jnp.where(iota == ik, -1.0, vals)
    scores_ref[...] = jnp.concatenate(scs, axis=1)
    idx_ref[...] = jnp.concatenate(ids, axis=1).astype(jnp.int32)


def _tc_topk(x, W):
    nblk = _TOKENS // _TOK_BLOCK
    return pl.pallas_call(
        _topk_body,
        grid=(nblk,),
        in_specs=[
            pl.BlockSpec((_TOK_BLOCK, _DIM), lambda i: (i, 0)),
            pl.BlockSpec((_NE, _DIM), lambda i: (0, 0)),
        ],
        out_specs=[
            pl.BlockSpec((_TOK_BLOCK, _K), lambda i: (i, 0)),
            pl.BlockSpec((_TOK_BLOCK, _K), lambda i: (i, 0)),
        ],
        out_shape=[
            jax.ShapeDtypeStruct((_TOKENS, _K), jnp.float32),
            jax.ShapeDtypeStruct((_TOKENS, _K), jnp.int32),
        ],
    )(x, W)


def _hist_body(keys_hbm, h_out, s_out, keys_v, hist_v, svec_v):
    c = lax.axis_index("c")
    s = lax.axis_index("s")
    w = s * _NC + c
    pltpu.sync_copy(keys_hbm.at[pl.ds(w * _CHUNK, _CHUNK)], keys_v)

    zeros16 = jnp.zeros((_L,), jnp.int32)
    for j in range(_VC * _NE * _L // _L):
        hist_v[pl.ds(j * _L, _L)] = zeros16

    lane = lax.broadcasted_iota(jnp.int32, (_L,), 0)
    base_idx = lane * _PLV
    base_hist = lane * _NE
    ones = jnp.ones((_L,), jnp.int32)

    def body(t, carry):
        for u in range(_VC):
            idx = base_idx + (u * _VCHUNK + t)
            k = plsc.load_gather(keys_v, [idx])
            plsc.addupdate_scatter(hist_v, [(u * _NE * _L) + base_hist + k],
                                   ones)
        return carry

    lax.fori_loop(0, _PLV, body, 0)

    for u in range(_VC):
        for j in range(_NE // _L):
            acc = zeros16
            for l in range(_L):
                acc = acc + hist_v[pl.ds(u * _NE * _L + l * _NE + j * _L, _L)]
            svec_v[pl.ds(u * _NE + j * _L, _L)] = acc

    pltpu.sync_copy(hist_v, h_out.at[pl.ds(w * _VC * _NE * _L, _VC * _NE * _L)])
    pltpu.sync_copy(svec_v, s_out.at[pl.ds(w * _VC * _NE, _VC * _NE)])


def _sc_hist(keys):
    mesh = plsc.VectorSubcoreMesh(core_axis_name="c", subcore_axis_name="s")
    f = pl.kernel(
        _hist_body,
        out_type=(
            jax.ShapeDtypeStruct((_NV * _NE * _L,), jnp.int32),
            jax.ShapeDtypeStruct((_NV * _NE,), jnp.int32),
        ),
        mesh=mesh,
        scratch_types=[
            pltpu.VMEM((_CHUNK,), jnp.int32),
            pltpu.VMEM((_VC * _NE * _L,), jnp.int32),
            pltpu.VMEM((_VC * _NE,), jnp.int32),
        ],
        compiler_params=pltpu.CompilerParams(needs_layout_passes=False),
    )
    return f(keys)


_VCP = 4
_CHUNK_P = _FLAT // _NS


def _place_body(keys_hbm, scores_hbm, h_hbm, s_hbm,
                scores_out, tok_out, hist_out,
                keys_v, scores_v, outpos_v, tok_v, sall_v, h_v,
                counters_v, hist64_v, shs_sp, sht_sp):
    c = lax.axis_index("c")
    s = lax.axis_index("s")

    if True:
        pltpu.sync_copy(keys_hbm.at[pl.ds(s * _CHUNK_P, _CHUNK_P)], keys_v)

        @pl.when(c == 1)
        def _():
            pltpu.sync_copy(scores_hbm.at[pl.ds(s * _CHUNK_P, _CHUNK_P)],
                            scores_v)
        pltpu.sync_copy(s_hbm, sall_v)
        pltpu.sync_copy(h_hbm.at[pl.ds(s * _VCP * _NE * _L, _VCP * _NE * _L)],
                        h_v)

        zeros16 = jnp.zeros((_L,), jnp.int32)

        T = []
        Cu = [[] for _ in range(_VCP)]
        for j in range(_NE // _L):
            def acc_body(v2, carry, j=j):
                tj, cj = carry
                row = sall_v[pl.ds(v2 * _NE + j * _L, _L)]
                tj = tj + row
                cj = cj + jnp.where(v2 < _VCP * s, row, zeros16)
                return (tj, cj)
            tj, cj = lax.fori_loop(0, _NV, acc_body, (zeros16, zeros16))
            T.append(tj)
            Cu[0].append(cj)
            for u in range(1, _VCP):
                cj = cj + sall_v[pl.ds((_VCP * s + u - 1) * _NE + j * _L, _L)]
                Cu[u].append(cj)

        G = []
        carry = jnp.zeros((), jnp.int32)
        for j in range(_NE // _L):
            cum = plsc.cumsum(T[j])
            G.append(cum - T[j] + carry)
            carry = carry + jnp.sum(T[j])

        @pl.when((c == 0) & (s == 0))
        def _():
            for j in range(_NE // _L):
                hist64_v[pl.ds(j * _L, _L)] = T[j]
            pltpu.sync_copy(hist64_v, hist_out)

        for u in range(_VCP):
            acc = [G[j] + Cu[u][j] for j in range(_NE // _L)]
            for l in range(_L):
                for j in range(_NE // _L):
                    off = u * _NE * _L + l * _NE + j * _L
                    counters_v[pl.ds(off, _L)] = acc[j]
                    acc[j] = acc[j] + h_v[pl.ds(off, _L)]

        lane = lax.broadcasted_iota(jnp.int32, (_L,), 0)
        base_idx = lane * _PLV
        base_hist = lane * _NE
        ones = jnp.ones((_L,), jnp.int32)
        gbase = s * _CHUNK_P

        def body(t, carry):
            for u in range(_VCP):
                idx = base_idx + (u * _VCHUNK + t)
                k = plsc.load_gather(keys_v, [idx])
                cidx = (u * _NE * _L) + base_hist + k
                pos = plsc.load_gather(counters_v, [cidx])
                plsc.store_scatter(outpos_v, [idx], pos)
                plsc.addupdate_scatter(counters_v, [cidx], ones)
            return carry

        lax.fori_loop(0, _PLV, body, 0)

        @pl.when(c == 0)
        def _():
            def tbody(t, carry):
                tok = lax.shift_right_logical(gbase + t * _L + lane, 3)
                tok_v[pl.ds(t * _L, _L)] = tok
                return carry

            lax.fori_loop(0, _CHUNK_P // _L, tbody, 0)
            pltpu.sync_copy(tok_v, sht_sp.at[outpos_v])
            plsc.subcore_barrier()
            pltpu.sync_copy(sht_sp.at[pl.ds(s * _CHUNK_P, _CHUNK_P)],
                            tok_out.at[pl.ds(s * _CHUNK_P, _CHUNK_P)])

        @pl.when(c == 1)
        def _():
            pltpu.sync_copy(scores_v, shs_sp.at[outpos_v])
            plsc.subcore_barrier()
            pltpu.sync_copy(shs_sp.at[pl.ds(s * _CHUNK_P, _CHUNK_P)],
                            scores_out.at[pl.ds(s * _CHUNK_P, _CHUNK_P)])


def _sc_place(keys, scores_flat, h, s):
    mesh = plsc.VectorSubcoreMesh(core_axis_name="c", subcore_axis_name="s")
    f = pl.kernel(
        _place_body,
        out_type=(
            jax.ShapeDtypeStruct((_FLAT,), jnp.float32),
            jax.ShapeDtypeStruct((_FLAT,), jnp.int32),
            jax.ShapeDtypeStruct((_NE,), jnp.int32),
        ),
        mesh=mesh,
        scratch_types=[
            pltpu.VMEM((_CHUNK_P,), jnp.int32),
            pltpu.VMEM((_CHUNK_P,), jnp.float32),
            pltpu.VMEM((_CHUNK_P,), jnp.int32),
            pltpu.VMEM((_CHUNK_P,), jnp.int32),
            pltpu.VMEM((_NV * _NE,), jnp.int32),
            pltpu.VMEM((_VCP * _NE * _L,), jnp.int32),
            pltpu.VMEM((_VCP * _NE * _L,), jnp.int32),
            pltpu.VMEM((_NE,), jnp.int32),
            pltpu.VMEM_SHARED((_FLAT,), jnp.float32),
            pltpu.VMEM_SHARED((_FLAT,), jnp.int32),
        ],
        compiler_params=pltpu.CompilerParams(needs_layout_passes=False),
    )
    return f(keys, scores_flat, h, s)


def kernel(x, W):
    top_scores, sel = _tc_topk(x, W)
    keys = sel.reshape(-1)
    scores_flat = top_scores.reshape(-1)
    h, s = _sc_hist(keys)
    out_scores, out_tok, hist = _sc_place(keys, scores_flat, h, s)
    return out_scores, out_tok, hist
